# Initial kernel scaffold; baseline (speedup 1.0000x reference)
import jax
import jax.numpy as jnp
from jax.experimental import pallas as pl


def kernel(x, fc1_w, fc1_b, bn_g, bn_b, sa_conv1_w, sa_conv1_b, sa_bn1_g, sa_bn1_b, sa_conv2_w, sa_conv2_b, sa_bn2_g, sa_bn2_b, bn2_g, bn2_b, fc4_w, fc4_b):
    raise NotImplementedError("write your pallas kernel here")


# speedup vs baseline: 7.0651x; 7.0651x over previous
import jax
import jax.numpy as jnp
from jax import lax
from jax.experimental import pallas as pl

RADIUS2 = 1.0
NSAMPLE = 4
N_FREQS = 10
EPS = 1e-5

ROWS_PER_BLOCK = 256

_HI = lax.Precision.HIGHEST


def _bf(a):
    return a.astype(jnp.bfloat16).astype(jnp.float32)


def _prep_body(xt_ref, fc1_wt_ref, fc1_b_ref, bn_g_ref, bn_b_ref,
               wf1_ref, wf2_ref, q_ref):
    xt = xt_ref[...]
    fc1_wt = _bf(fc1_wt_ref[...])
    bn = xt.shape[1]
    half = bn // 2

    dn = (((1,), (0,)), ((), ()))
    acc = lax.dot_general(fc1_wt[:, 0:2], _bf(xt), dn, precision=_HI,
                          preferred_element_type=jnp.float32)
    for i in range(N_FREQS):
        f = 2.0 ** i
        acc = acc + lax.dot_general(fc1_wt[:, 2 + 4 * i:4 + 4 * i],
                                    _bf(jnp.sin(f * xt)), dn, precision=_HI,
                                    preferred_element_type=jnp.float32)
        acc = acc + lax.dot_general(fc1_wt[:, 4 + 4 * i:6 + 4 * i],
                                    _bf(jnp.cos(f * xt)), dn, precision=_HI,
                                    preferred_element_type=jnp.float32)
    h = jax.nn.relu(acc + fc1_b_ref[...])

    mean = jnp.mean(h, axis=1, keepdims=True)
    d = h - mean
    var = jnp.mean(d * d, axis=1, keepdims=True)
    hbn = (bn_g_ref[...] * d) / jnp.sqrt(var + EPS) + bn_b_ref[...]

    hg0 = jnp.max(hbn[:, :half], axis=1, keepdims=True)
    hg1 = jnp.max(hbn[:, half:], axis=1, keepdims=True)

    wf2 = _bf(wf2_ref[...])
    g0 = lax.dot_general(wf2, _bf(hg0), dn, precision=_HI,
                         preferred_element_type=jnp.float32)
    g1 = lax.dot_general(wf2, _bf(hg1), dn, precision=_HI,
                         preferred_element_type=jnp.float32)
    gterm = jnp.concatenate([jnp.broadcast_to(g0, (g0.shape[0], half)),
                             jnp.broadcast_to(g1, (g1.shape[0], half))], axis=1)

    q_ref[...] = lax.dot_general(_bf(wf1_ref[...]), _bf(hbn), dn, precision=_HI,
                                 preferred_element_type=jnp.float32) + gterm


def _scan_body(x_ref, xt_ref, xtc_ref, q_ref, wxt_ref, b1_ref,
               z0_ref, z1_ref, z2_ref, z3_ref):
    xi = x_ref[0]
    xtb = xt_ref[0]
    xtc = xtc_ref[0]
    qb = q_ref[...]
    wxt = wxt_ref[...]
    b1 = b1_ref[...]
    n = xtb.shape[1]
    r = xi.shape[0]

    si = jnp.sum(xi * xi, axis=1, keepdims=True)
    sj = jnp.sum(xtb * xtb, axis=0, keepdims=True)
    xih = _bf(xi)
    xbh = _bf(xtb)
    dot = (xih[:, 0:1] * xbh[0:1, :]) + (xih[:, 1:2] * xbh[1:2, :])
    dist = (si + sj) - 2.0 * dot

    cols = lax.broadcasted_iota(jnp.int32, (r, n), 1).astype(jnp.float32)
    sentinel = jnp.float32(n)
    cand = jnp.where(dist <= RADIUS2, cols, sentinel)

    out_refs = (z0_ref, z1_ref, z2_ref, z3_ref)
    j0 = None
    for k in range(NSAMPLE):
        jk = jnp.min(cand, axis=1, keepdims=True)
        if k == 0:
            j0 = jk
            jsel = jk
        else:
            jsel = jnp.where(jk >= sentinel, j0, jk)
        sel = (cols == jsel).astype(jnp.float32)
        qg = lax.dot_general(qb, sel, (((1,), (1,)), ((), ())), precision=_HI,
                             preferred_element_type=jnp.float32)
        xg = lax.dot_general(xtb, sel, (((1,), (1,)), ((), ())), precision=_HI,
                             preferred_element_type=jnp.float32)
        dxy = _bf(xg - xtc)
        xyzterm = (wxt[:, 0:1] * dxy[0:1, :]) + (wxt[:, 1:2] * dxy[1:2, :])
        out_refs[k][...] = (qg + xyzterm) + b1
        cand = jnp.where(cand == jk, sentinel, cand)


def _head_body(z0_ref, z1_ref, z2_ref, z3_ref,
               g1_ref, b1_ref, w2t_ref, b2_ref, g2_ref, bb2_ref,
               g3_ref, b3_ref, fc4_wt_ref, fc4_b_ref, out_ref):
    zs = [z0_ref[...], z1_ref[...], z2_ref[...], z3_ref[...]]
    cnt = jnp.float32(zs[0].shape[1] * NSAMPLE)
    dn = (((1,), (0,)), ((), ()))

    s = sum(jnp.sum(z, axis=1, keepdims=True) for z in zs)
    mean1 = s / cnt
    ssd = sum(jnp.sum((z - mean1) * (z - mean1), axis=1, keepdims=True)
              for z in zs)
    rstd1 = jnp.sqrt(ssd / cnt + EPS)
    g1 = g1_ref[...]
    b1 = b1_ref[...]

    w2t = _bf(w2t_ref[...])
    b2 = b2_ref[...]
    z2s = [lax.dot_general(
               w2t, _bf(jax.nn.relu((g1 * (z - mean1)) / rstd1 + b1)), dn,
               precision=_HI, preferred_element_type=jnp.float32) + b2
           for z in zs]

    s2 = sum(jnp.sum(z, axis=1, keepdims=True) for z in z2s)
    mean2 = s2 / cnt
    ssd2 = sum(jnp.sum((z - mean2) * (z - mean2), axis=1, keepdims=True)
               for z in z2s)
    rstd2 = jnp.sqrt(ssd2 / cnt + EPS)
    g2 = g2_ref[...]
    bb2 = bb2_ref[...]

    a2s = [jax.nn.relu((g2 * (z - mean2)) / rstd2 + bb2) for z in z2s]
    m = jnp.maximum(jnp.maximum(a2s[0], a2s[1]), jnp.maximum(a2s[2], a2s[3]))

    mean3 = jnp.mean(m, axis=1, keepdims=True)
    d3 = m - mean3
    var3 = jnp.mean(d3 * d3, axis=1, keepdims=True)
    t = jax.nn.relu((g3_ref[...] * d3) / jnp.sqrt(var3 + EPS) + b3_ref[...])
    out_ref[...] = lax.dot_general(_bf(fc4_wt_ref[...]), _bf(t), dn,
                                   precision=_HI,
                                   preferred_element_type=jnp.float32) + fc4_b_ref[...]


def kernel(x, fc1_w, fc1_b, bn_g, bn_b, sa_conv1_w, sa_conv1_b,
           sa_bn1_g, sa_bn1_b, sa_conv2_w, sa_conv2_b, sa_bn2_g, sa_bn2_b,
           bn2_g, bn2_b, fc4_w, fc4_b):
    B, N, _ = x.shape
    BN = B * N
    R = ROWS_PER_BLOCK
    nblk = N // R

    xtf = x.reshape(BN, 2).T
    xt = jnp.transpose(x, (0, 2, 1))

    wf1_t = sa_conv1_w[2:62, :].T
    wf2_t = sa_conv1_w[62:122, :].T

    q = pl.pallas_call(
        _prep_body,
        out_shape=jax.ShapeDtypeStruct((30, BN), jnp.float32),
    )(xtf, fc1_w.T, fc1_b.reshape(60, 1), bn_g.reshape(60, 1),
      bn_b.reshape(60, 1), wf1_t, wf2_t)

    zs = pl.pallas_call(
        _scan_body,
        grid=(B, nblk),
        in_specs=[
            pl.BlockSpec((1, R, 2), lambda b, i: (b, i, 0)),
            pl.BlockSpec((1, 2, N), lambda b, i: (b, 0, 0)),
            pl.BlockSpec((1, 2, R), lambda b, i: (b, 0, i)),
            pl.BlockSpec((30, N), lambda b, i: (0, b)),
            pl.BlockSpec((30, 2), lambda b, i: (0, 0)),
            pl.BlockSpec((30, 1), lambda b, i: (0, 0)),
        ],
        out_specs=[pl.BlockSpec((30, R), lambda b, i: (0, b * nblk + i))] * NSAMPLE,
        out_shape=[jax.ShapeDtypeStruct((30, BN), jnp.float32)] * NSAMPLE,
    )(x, xt, xt, q, sa_conv1_w[0:2, :].T, sa_conv1_b.reshape(30, 1))

    out = pl.pallas_call(
        _head_body,
        out_shape=jax.ShapeDtypeStruct((2, BN), jnp.float32),
    )(*zs,
      sa_bn1_g.reshape(30, 1), sa_bn1_b.reshape(30, 1),
      sa_conv2_w.T, sa_conv2_b.reshape(60, 1),
      sa_bn2_g.reshape(60, 1), sa_bn2_b.reshape(60, 1),
      bn2_g.reshape(60, 1), bn2_b.reshape(60, 1),
      fc4_w.T, fc4_b.reshape(2, 1))

    return out.reshape(2, B, N).transpose(1, 2, 0)
